# baseline (speedup 1.0000x reference)
import functools

import jax
import jax.numpy as jnp
from jax import lax
from jax.experimental import pallas as pl
from jax.experimental.pallas import tpu as pltpu
from jax.experimental.pallas import tpu_sc as plsc

_B, _S, _H, _E, _K = 1, 2048, 1024, 16, 2
_CAP = 384
_TS = 256
_GRID = _S // _TS

_NC, _NS = 2, 16
_NW = _NC * _NS
_TPW = _S // _NW
_CHUNK = 16
_NCHUNK = _TPW // _CHUNK
_ROW = _E * _CAP
_BUF = _CHUNK * _ROW


def _mlp_body(x_ref, w1_ref, b1_ref, w2_ref, b2_ref,
              probs_ref, disp0_ref, comb0_ref, aux_ref, acc_ref):
    x = x_ref[...]
    h = jnp.dot(x, w1_ref[...], preferred_element_type=jnp.float32)
    h = jnp.maximum(h + b1_ref[...], 0.0)
    logits = jnp.dot(h, w2_ref[...], preferred_element_type=jnp.float32)
    logits = logits + b2_ref[...]

    m = jnp.max(logits, axis=1, keepdims=True)
    ex = jnp.exp(logits - m)
    p = ex / jnp.sum(ex, axis=1, keepdims=True)
    probs_ref[...] = p

    idx = lax.broadcasted_iota(jnp.int32, (_TS, _E), 1)
    m1 = jnp.max(p, axis=1, keepdims=True)
    i1 = jnp.min(jnp.where(p == m1, idx, _E), axis=1, keepdims=True)
    mask1 = idx == i1
    pm = jnp.where(mask1, -1.0, p)
    m2 = jnp.max(pm, axis=1, keepdims=True)
    i2 = jnp.min(jnp.where(pm == m2, idx, _E), axis=1, keepdims=True)
    mask2 = idx == i2
    denom = m1 + m2
    comb0_ref[...] = (jnp.where(mask1, m1, 0.0) + jnp.where(mask2, m2, 0.0)) / denom
    disp0_ref[...] = (mask1 | mask2).astype(jnp.float32)

    step = pl.program_id(0)
    psum = jnp.sum(p, axis=0, keepdims=True)

    @pl.when(step == 0)
    def _():
        acc_ref[...] = psum

    @pl.when(step != 0)
    def _():
        acc_ref[...] = acc_ref[...] + psum

    @pl.when(step == _GRID - 1)
    def _():
        rp = acc_ref[...] / (_B * _S)
        aux_ref[0, 0] = jnp.sum(rp * jnp.log(rp * _E + 1e-09))


_mlp_call = pl.pallas_call(
    _mlp_body,
    grid=(_GRID,),
    in_specs=[
        pl.BlockSpec((_TS, _H), lambda i: (i, 0)),
        pl.BlockSpec((_H, _H), lambda i: (0, 0)),
        pl.BlockSpec((1, _H), lambda i: (0, 0)),
        pl.BlockSpec((_H, _E), lambda i: (0, 0)),
        pl.BlockSpec((1, _E), lambda i: (0, 0)),
    ],
    out_specs=[
        pl.BlockSpec((_TS, _E), lambda i: (i, 0)),
        pl.BlockSpec((_TS, _E), lambda i: (i, 0)),
        pl.BlockSpec((_TS, _E), lambda i: (i, 0)),
        pl.BlockSpec((1, 1), lambda i: (0, 0), memory_space=pltpu.SMEM),
    ],
    out_shape=[
        jax.ShapeDtypeStruct((_S, _E), jnp.float32),
        jax.ShapeDtypeStruct((_S, _E), jnp.float32),
        jax.ShapeDtypeStruct((_S, _E), jnp.float32),
        jax.ShapeDtypeStruct((1, 1), jnp.float32),
    ],
    scratch_shapes=[pltpu.VMEM((1, _E), jnp.float32)],
)


def _disp_body(d0_ref, out_ref):
    cap0 = lax.broadcasted_iota(jnp.int32, (_TS, _E, _CAP), 2) == 0
    out_ref[...] = jnp.where(cap0, d0_ref[...][:, :, None], 0.0)


_disp_call = pl.pallas_call(
    _disp_body,
    grid=(_GRID,),
    in_specs=[pl.BlockSpec((_TS, _E), lambda i: (i, 0))],
    out_specs=[pl.BlockSpec((_TS, _E, _CAP), lambda i: (i, 0, 0))],
    out_shape=[jax.ShapeDtypeStruct((_S, _E, _CAP), jnp.float32)],
)


@functools.partial(
    pl.kernel,
    mesh=plsc.VectorSubcoreMesh(core_axis_name="c", subcore_axis_name="s"),
    out_type=jax.ShapeDtypeStruct((_S * _ROW,), jnp.float32),
    compiler_params=pltpu.CompilerParams(needs_layout_passes=False),
    scratch_types=[
        pltpu.VMEM((_BUF,), jnp.float32),
        pltpu.VMEM((_TPW * _E,), jnp.float32),
    ],
)
def _sc_combine(comb0_hbm, zeros_hbm, out_hbm, buf_v, vals_v):
    wid = lax.axis_index("s") * _NC + lax.axis_index("c")
    base = wid * _TPW
    pltpu.sync_copy(comb0_hbm.at[pl.ds(base * _E, _TPW * _E)], vals_v)
    pltpu.sync_copy(zeros_hbm, buf_v)
    for c in range(_NCHUNK):
        for t in range(_CHUNK):
            v = vals_v[pl.ds((c * _CHUNK + t) * _E, _E)]
            pos = lax.iota(jnp.int32, _E) * _CAP + t * _ROW
            plsc.store_scatter(buf_v, [pos], v)
        off = (base + c * _CHUNK) * _ROW
        pltpu.sync_copy(buf_v, out_hbm.at[pl.ds(off, _BUF)])


def kernel(hidden_states, W1, b1, W2, b2):
    x = hidden_states.reshape(_S, _H)
    probs, disp0, comb0, aux = _mlp_call(
        x, W1, b1.reshape(1, _H), W2, b2.reshape(1, _E))
    dispatch, = _disp_call(disp0)
    combine = _sc_combine(comb0.reshape(_S * _E),
                          jnp.zeros((_BUF,), jnp.float32))
    return (dispatch.reshape(_B, _S, _E, _CAP),
            combine.reshape(_B, _S, _E, _CAP),
            probs.reshape(_B, _S, _E),
            aux[0, 0])


# speedup vs baseline: 1.6138x; 1.6138x over previous
import functools

import jax
import jax.numpy as jnp
from jax import lax
from jax.experimental import pallas as pl
from jax.experimental.pallas import tpu as pltpu
from jax.experimental.pallas import tpu_sc as plsc

_B, _S, _H, _E, _K = 1, 2048, 1024, 16, 2
_CAP = 384
_TS = 256
_GRID = _S // _TS

_NC, _NS = 2, 16
_NW = _NC * _NS
_TPW = _S // _NW
_CHUNK = 16
_NCHUNK = _TPW // _CHUNK
_ROW = _E * _CAP
_BUF = _CHUNK * _ROW


def _mlp_body(x_ref, w1_ref, b1_ref, w2_ref, b2_ref,
              probs_ref, disp0_ref, comb0_ref, aux_ref, acc_ref):
    x = x_ref[...]
    h = jnp.dot(x, w1_ref[...], preferred_element_type=jnp.float32)
    h = jnp.maximum(h + b1_ref[...], 0.0)
    logits = jnp.dot(h, w2_ref[...], preferred_element_type=jnp.float32)
    logits = logits + b2_ref[...]

    m = jnp.max(logits, axis=1, keepdims=True)
    ex = jnp.exp(logits - m)
    p = ex / jnp.sum(ex, axis=1, keepdims=True)
    probs_ref[...] = p

    idx = lax.broadcasted_iota(jnp.int32, (_TS, _E), 1)
    m1 = jnp.max(p, axis=1, keepdims=True)
    i1 = jnp.min(jnp.where(p == m1, idx, _E), axis=1, keepdims=True)
    mask1 = idx == i1
    pm = jnp.where(mask1, -1.0, p)
    m2 = jnp.max(pm, axis=1, keepdims=True)
    i2 = jnp.min(jnp.where(pm == m2, idx, _E), axis=1, keepdims=True)
    mask2 = idx == i2
    denom = m1 + m2
    comb0_ref[...] = (jnp.where(mask1, m1, 0.0) + jnp.where(mask2, m2, 0.0)) / denom
    disp0_ref[...] = (mask1 | mask2).astype(jnp.float32)

    step = pl.program_id(0)
    psum = jnp.sum(p, axis=0, keepdims=True)

    @pl.when(step == 0)
    def _():
        acc_ref[...] = psum

    @pl.when(step != 0)
    def _():
        acc_ref[...] = acc_ref[...] + psum

    @pl.when(step == _GRID - 1)
    def _():
        rp = acc_ref[...] / (_B * _S)
        aux_ref[0, 0] = jnp.sum(rp * jnp.log(rp * _E + 1e-09))


_mlp_call = pl.pallas_call(
    _mlp_body,
    grid=(_GRID,),
    in_specs=[
        pl.BlockSpec((_TS, _H), lambda i: (i, 0)),
        pl.BlockSpec((_H, _H), lambda i: (0, 0)),
        pl.BlockSpec((1, _H), lambda i: (0, 0)),
        pl.BlockSpec((_H, _E), lambda i: (0, 0)),
        pl.BlockSpec((1, _E), lambda i: (0, 0)),
    ],
    out_specs=[
        pl.BlockSpec((_TS, _E), lambda i: (i, 0)),
        pl.BlockSpec((_TS, _E), lambda i: (i, 0)),
        pl.BlockSpec((_TS, _E), lambda i: (i, 0)),
        pl.BlockSpec((1, 1), lambda i: (0, 0), memory_space=pltpu.SMEM),
    ],
    out_shape=[
        jax.ShapeDtypeStruct((_S, _E), jnp.float32),
        jax.ShapeDtypeStruct((_S, _E), jnp.float32),
        jax.ShapeDtypeStruct((_S, _E), jnp.float32),
        jax.ShapeDtypeStruct((1, 1), jnp.float32),
    ],
    scratch_shapes=[pltpu.VMEM((1, _E), jnp.float32)],
)


def _disp_body(d0_ref, out_ref):
    cap0 = lax.broadcasted_iota(jnp.int32, (_TS, _E, _CAP), 2) == 0
    out_ref[...] = jnp.where(cap0, d0_ref[...][:, :, None], 0.0)


_disp_call = pl.pallas_call(
    _disp_body,
    grid=(_GRID,),
    in_specs=[pl.BlockSpec((_TS, _E), lambda i: (i, 0))],
    out_specs=[pl.BlockSpec((_TS, _E, _CAP), lambda i: (i, 0, 0))],
    out_shape=[jax.ShapeDtypeStruct((_S, _E, _CAP), jnp.float32)],
)


@functools.partial(
    pl.kernel,
    mesh=plsc.VectorSubcoreMesh(core_axis_name="c", subcore_axis_name="s"),
    out_type=jax.ShapeDtypeStruct((_S, _E, _CAP), jnp.float32),
    compiler_params=pltpu.CompilerParams(needs_layout_passes=False),
    scratch_types=[
        pltpu.VMEM((_CHUNK, _E, _CAP), jnp.float32),
        pltpu.VMEM((_TPW * _E,), jnp.float32),
    ],
)
def _sc_combine(comb0_hbm, zeros_hbm, out_hbm, buf_v, vals_v):
    wid = lax.axis_index("s") * _NC + lax.axis_index("c")
    base = wid * _TPW
    pltpu.sync_copy(comb0_hbm.at[pl.ds(base * _E, _TPW * _E)], vals_v)
    pltpu.sync_copy(zeros_hbm, buf_v)
    pos_e = lax.iota(jnp.int32, _E)
    pos_c = jnp.zeros((_E,), jnp.int32)
    for c in range(_NCHUNK):
        for t in range(_CHUNK):
            v = vals_v[pl.ds((c * _CHUNK + t) * _E, _E)]
            pos_t = jnp.full((_E,), t, jnp.int32)
            plsc.store_scatter(buf_v, [pos_t, pos_e, pos_c], v)
        pltpu.sync_copy(buf_v, out_hbm.at[pl.ds(base + c * _CHUNK, _CHUNK)])


def kernel(hidden_states, W1, b1, W2, b2):
    x = hidden_states.reshape(_S, _H)
    probs, disp0, comb0, aux = _mlp_call(
        x, W1, b1.reshape(1, _H), W2, b2.reshape(1, _E))
    combine = _sc_combine(comb0.reshape(_S * _E),
                          jnp.zeros((_CHUNK, _E, _CAP), jnp.float32))
    dispatch, = _disp_call(disp0)
    return (dispatch.reshape(_B, _S, _E, _CAP),
            combine.reshape(_B, _S, _E, _CAP),
            probs.reshape(_B, _S, _E),
            aux[0, 0])


# speedup vs baseline: 3.1066x; 1.9250x over previous
import jax
import jax.numpy as jnp
from jax import lax
from jax.experimental import pallas as pl
from jax.experimental.pallas import tpu as pltpu

_B, _S, _H, _E, _K = 1, 2048, 1024, 16, 2
_CAP = 384
_TS = 256
_GRID = _S // _TS


def _router_body(x_ref, w1_ref, b1_ref, w2_ref, b2_ref,
                 disp_ref, comb_ref, probs_ref, aux_ref, acc_ref):
    x = x_ref[...]
    h = jnp.dot(x, w1_ref[...], preferred_element_type=jnp.float32)
    h = jnp.maximum(h + b1_ref[...], 0.0)
    logits = jnp.dot(h, w2_ref[...], preferred_element_type=jnp.float32)
    logits = logits + b2_ref[...]

    m = jnp.max(logits, axis=1, keepdims=True)
    ex = jnp.exp(logits - m)
    p = ex / jnp.sum(ex, axis=1, keepdims=True)
    probs_ref[...] = p

    idx = lax.broadcasted_iota(jnp.int32, (_TS, _E), 1)
    m1 = jnp.max(p, axis=1, keepdims=True)
    i1 = jnp.min(jnp.where(p == m1, idx, _E), axis=1, keepdims=True)
    mask1 = idx == i1
    pm = jnp.where(mask1, -1.0, p)
    m2 = jnp.max(pm, axis=1, keepdims=True)
    i2 = jnp.min(jnp.where(pm == m2, idx, _E), axis=1, keepdims=True)
    mask2 = idx == i2
    denom = m1 + m2
    comb0 = (jnp.where(mask1, m1, 0.0) + jnp.where(mask2, m2, 0.0)) / denom
    disp0 = (mask1 | mask2).astype(jnp.float32)

    cap0 = lax.broadcasted_iota(jnp.int32, (_TS, _E, _CAP), 2) == 0
    disp_ref[...] = jnp.where(cap0, disp0[:, :, None], 0.0)
    comb_ref[...] = jnp.where(cap0, comb0[:, :, None], 0.0)

    step = pl.program_id(0)
    psum = jnp.sum(p, axis=0, keepdims=True)

    @pl.when(step == 0)
    def _():
        acc_ref[...] = psum

    @pl.when(step != 0)
    def _():
        acc_ref[...] = acc_ref[...] + psum

    @pl.when(step == _GRID - 1)
    def _():
        rp = acc_ref[...] / (_B * _S)
        aux_ref[0, 0] = jnp.sum(rp * jnp.log(rp * _E + 1e-09))


_call = pl.pallas_call(
    _router_body,
    grid=(_GRID,),
    in_specs=[
        pl.BlockSpec((_TS, _H), lambda i: (i, 0)),
        pl.BlockSpec((_H, _H), lambda i: (0, 0)),
        pl.BlockSpec((1, _H), lambda i: (0, 0)),
        pl.BlockSpec((_H, _E), lambda i: (0, 0)),
        pl.BlockSpec((1, _E), lambda i: (0, 0)),
    ],
    out_specs=[
        pl.BlockSpec((_TS, _E, _CAP), lambda i: (i, 0, 0)),
        pl.BlockSpec((_TS, _E, _CAP), lambda i: (i, 0, 0)),
        pl.BlockSpec((_TS, _E), lambda i: (i, 0)),
        pl.BlockSpec((1, 1), lambda i: (0, 0), memory_space=pltpu.SMEM),
    ],
    out_shape=[
        jax.ShapeDtypeStruct((_S, _E, _CAP), jnp.float32),
        jax.ShapeDtypeStruct((_S, _E, _CAP), jnp.float32),
        jax.ShapeDtypeStruct((_S, _E), jnp.float32),
        jax.ShapeDtypeStruct((1, 1), jnp.float32),
    ],
    scratch_shapes=[pltpu.VMEM((1, _E), jnp.float32)],
)


def kernel(hidden_states, W1, b1, W2, b2):
    x = hidden_states.reshape(_S, _H)
    disp, comb, probs, aux = _call(x, W1, b1.reshape(1, _H), W2, b2.reshape(1, _E))
    return (disp.reshape(_B, _S, _E, _CAP),
            comb.reshape(_B, _S, _E, _CAP),
            probs.reshape(_B, _S, _E),
            aux[0, 0])
